# baseline (speedup 1.0000x reference)
import functools

import jax
import jax.numpy as jnp
from jax import lax
from jax.experimental import pallas as pl
from jax.experimental.pallas import tpu as pltpu
from jax.experimental.pallas import tpu_sc as plsc

_NC, _NS = 2, 16
_NW = _NC * _NS
_CHUNK = 128
_PAD = 16


def _seg_body(n, d, n_chunks, iters, table, ei2, z_d, sum_out,
              ei_v, rows_v, acc_sh, sem0, sem1, sem_c=None,
              z_c=None, ones_h=None, cnt_out=None, ones_v=None, cnt_sh=None):
    c = lax.axis_index("c")
    s = lax.axis_index("s")
    wid = s * _NC + c
    rpt = (n // (_NS * 8)) * 8
    tail = n + _PAD - _NS * rpt
    base_row = s * rpt
    sems = (sem0, sem1)

    pltpu.sync_copy(z_d, acc_sh.at[pl.ds(base_row, rpt)])
    if cnt_sh is not None:
        pltpu.sync_copy(z_c, cnt_sh.at[pl.ds(base_row, rpt)])
        pltpu.sync_copy(ones_h, ones_v)

    @pl.when(s == 0)
    def _():
        pltpu.sync_copy(z_d.at[pl.ds(0, tail)],
                        acc_sh.at[pl.ds(_NS * rpt, tail)])
        if cnt_sh is not None:
            pltpu.sync_copy(z_c.at[pl.ds(0, tail)],
                            cnt_sh.at[pl.ds(_NS * rpt, tail)])

    plsc.subcore_barrier()

    pltpu.sync_copy(ei2.at[wid], ei_v.at[0])
    pltpu.async_copy(table.at[ei_v.at[0, 0]], rows_v.at[0], sem0)

    @pl.loop(0, iters, step=4)
    def _(i):
        for b in range(4):
            j = i + b
            cid_nxt = wid + (j + 1) * _NW

            @pl.when(jnp.logical_and(j + 1 < iters, cid_nxt < n_chunks))
            def _():
                pltpu.sync_copy(ei2.at[cid_nxt], ei_v.at[(b + 1) % 4])
                pltpu.async_copy(table.at[ei_v.at[(b + 1) % 4, 0]],
                                 rows_v.at[(b + 1) % 2], sems[(b + 1) % 2])

            @pl.when(wid + j * _NW < n_chunks)
            def _():
                pltpu.make_async_copy(table.at[pl.ds(0, _CHUNK)],
                                      rows_v.at[b % 2], sems[b % 2]).wait()
                pltpu.sync_copy(rows_v.at[b % 2], acc_sh.at[ei_v.at[b, 1]],
                                add=True)
                if cnt_sh is not None:
                    pltpu.async_copy(ones_v, cnt_sh.at[ei_v.at[b, 1]],
                                     sem_c, add=True)

    if cnt_sh is not None:
        done = (n_chunks - wid + _NW - 1) // _NW

        @pl.loop(0, done)
        def _(i):
            pltpu.make_async_copy(ones_v, cnt_sh.at[pl.ds(0, _CHUNK)],
                                  sem_c).wait()

    plsc.subcore_barrier()

    pltpu.sync_copy(acc_sh.at[pl.ds(base_row, rpt)],
                    sum_out.at[pl.ds(c * n + base_row, rpt)])
    if cnt_sh is not None:
        pltpu.sync_copy(cnt_sh.at[pl.ds(base_row, rpt)],
                        cnt_out.at[pl.ds(c * n + base_row, rpt)])

    @pl.when(s == 0)
    def _():
        pltpu.sync_copy(acc_sh.at[pl.ds(_NS * rpt, n - _NS * rpt)],
                        sum_out.at[pl.ds(c * n + _NS * rpt, n - _NS * rpt)])
        if cnt_sh is not None:
            pltpu.sync_copy(cnt_sh.at[pl.ds(_NS * rpt, n - _NS * rpt)],
                            cnt_out.at[pl.ds(c * n + _NS * rpt, n - _NS * rpt)])


def _grid(e):
    n_chunks = -(-e // _CHUNK)
    iters = -(-n_chunks // _NW)
    iters = -(-iters // 4) * 4
    return n_chunks, iters


@functools.lru_cache(maxsize=None)
def _make_segsum_count(n, e, d):
    n_chunks, iters = _grid(e)
    mesh = plsc.VectorSubcoreMesh(core_axis_name="c", subcore_axis_name="s")

    @functools.partial(
        pl.kernel,
        out_type=(jax.ShapeDtypeStruct((_NC * n, d), jnp.float32),
                  jax.ShapeDtypeStruct((_NC * n, 16), jnp.float32)),
        mesh=mesh,
        scratch_types=[
            pltpu.VMEM((4, 2, _CHUNK), jnp.int32),
            pltpu.VMEM((2, _CHUNK, d), jnp.float32),
            pltpu.VMEM((_CHUNK, 16), jnp.float32),
            pltpu.VMEM_SHARED((n + _PAD, d), jnp.float32),
            pltpu.VMEM_SHARED((n + _PAD, 16), jnp.float32),
            pltpu.SemaphoreType.DMA,
            pltpu.SemaphoreType.DMA,
            pltpu.SemaphoreType.DMA,
        ],
        compiler_params=pltpu.CompilerParams(use_tc_tiling_on_sc=False),
    )
    def seg(table, ei2, z_d, z_c, ones_h, sum_out, cnt_out,
            ei_v, rows_v, ones_v, acc_sh, cnt_sh, sem0, sem1, sem_c):
        _seg_body(n, d, n_chunks, iters, table, ei2, z_d, sum_out,
                  ei_v, rows_v, acc_sh, sem0, sem1, sem_c=sem_c,
                  z_c=z_c, ones_h=ones_h, cnt_out=cnt_out,
                  ones_v=ones_v, cnt_sh=cnt_sh)

    return seg


@functools.lru_cache(maxsize=None)
def _make_segsum(n, e, d):
    n_chunks, iters = _grid(e)
    mesh = plsc.VectorSubcoreMesh(core_axis_name="c", subcore_axis_name="s")

    @functools.partial(
        pl.kernel,
        out_type=jax.ShapeDtypeStruct((_NC * n, d), jnp.float32),
        mesh=mesh,
        scratch_types=[
            pltpu.VMEM((4, 2, _CHUNK), jnp.int32),
            pltpu.VMEM((2, _CHUNK, d), jnp.float32),
            pltpu.VMEM_SHARED((n + _PAD, d), jnp.float32),
            pltpu.SemaphoreType.DMA,
            pltpu.SemaphoreType.DMA,
        ],
        compiler_params=pltpu.CompilerParams(use_tc_tiling_on_sc=False),
    )
    def seg(table, ei2, z_d, sum_out, ei_v, rows_v, acc_sh, sem0, sem1):
        _seg_body(n, d, n_chunks, iters, table, ei2, z_d, sum_out,
                  ei_v, rows_v, acc_sh, sem0, sem1)

    return seg


def _mm_body(x_ref, w_ref, o_ref):
    o_ref[...] = jnp.dot(x_ref[...], w_ref[...],
                         preferred_element_type=jnp.float32)


def _matmul(x, w):
    return pl.pallas_call(
        _mm_body,
        out_shape=jax.ShapeDtypeStruct((x.shape[0], w.shape[1]), jnp.float32),
    )(x, w)


def _layer1(sums, cnts, xr, b):
    n = xr.shape[0]

    def body(s_ref, c_ref, xr_ref, b_ref, o_ref):
        sarr = s_ref[...]
        carr = c_ref[...]
        sm = sarr[:n] + sarr[n:]
        cnt = carr[:n, 0:1] + carr[n:, 0:1]
        o_ref[...] = jnp.maximum(sm / jnp.maximum(cnt, 1.0) + b_ref[...]
                                 + xr_ref[...], 0.0)

    return pl.pallas_call(
        body,
        out_shape=jax.ShapeDtypeStruct(xr.shape, jnp.float32),
    )(sums, cnts, xr, b)


def _layer2(sums, cnts, h, wl, wr, b):
    n = h.shape[0]

    def body(s_ref, c_ref, h_ref, wl_ref, wr_ref, b_ref, o_ref):
        sarr = s_ref[...]
        carr = c_ref[...]
        sm = sarr[:n] + sarr[n:]
        cnt = carr[:n, 0:1] + carr[n:, 0:1]
        a2 = sm / jnp.maximum(cnt, 1.0)
        o = (jnp.dot(a2, wl_ref[...], preferred_element_type=jnp.float32)
             + jnp.dot(h_ref[...], wr_ref[...],
                       preferred_element_type=jnp.float32)
             + b_ref[...])
        m = jnp.max(o, axis=1, keepdims=True)
        lse = jnp.log(jnp.sum(jnp.exp(o - m), axis=1, keepdims=True)) + m
        o_ref[...] = o - lse

    return pl.pallas_call(
        body,
        out_shape=jax.ShapeDtypeStruct((n, wl.shape[1]), jnp.float32),
    )(sums, cnts, h, wl, wr, b)


def kernel(x, edge_index, W1l, b1l, W1r, W2l, b2l, W2r):
    n, _ = x.shape
    d_hid = W1l.shape[1]
    e = edge_index.shape[1]
    src = edge_index[0]
    dst = edge_index[1]

    e_pad = -(-e // _CHUNK) * _CHUNK
    if e_pad != e:
        src = jnp.concatenate([src, jnp.zeros((e_pad - e,), jnp.int32)])
        dst = jnp.concatenate([dst, jnp.full((e_pad - e,), n, jnp.int32)])
    ei2 = jnp.stack([src.reshape(-1, _CHUNK), dst.reshape(-1, _CHUNK)], axis=1)

    xcat = _matmul(x, jnp.concatenate([W1l, W1r], axis=1))
    p = xcat[:, :d_hid]
    xr = xcat[:, d_hid:]

    rpt = (n // (_NS * 8)) * 8
    z_d = jnp.zeros((rpt, d_hid), jnp.float32)
    z_c = jnp.zeros((rpt, 16), jnp.float32)
    ones_h = jnp.ones((_CHUNK, 16), jnp.float32)

    sums1, cnts = _make_segsum_count(n, e, d_hid)(p, ei2, z_d, z_c, ones_h)
    h = _layer1(sums1, cnts, xr, b1l.reshape(1, -1))
    sums2 = _make_segsum(n, e, d_hid)(h, ei2, z_d)
    return _layer2(sums2, cnts, h, W2l, W2r, b2l.reshape(1, -1))


# speedup vs baseline: 1.0618x; 1.0618x over previous
import functools

import jax
import jax.numpy as jnp
from jax import lax
from jax.experimental import pallas as pl
from jax.experimental.pallas import tpu as pltpu
from jax.experimental.pallas import tpu_sc as plsc

_NC, _NS = 2, 16
_NW = _NC * _NS
_CHUNK = 128
_PAD = 16


def _seg_body(n, d, n_chunks, iters, table, ei2, z_d, sum_out,
              ei_v, rows_v, acc_sh, sem0, sem1, sem_c=None,
              z_c=None, ones_h=None, cnt_out=None, ones_v=None, cnt_sh=None):
    c = lax.axis_index("c")
    s = lax.axis_index("s")
    wid = s * _NC + c
    rpt = (n // (_NS * 8)) * 8
    tail = n + _PAD - _NS * rpt
    base_row = s * rpt
    sems = (sem0, sem1)

    pltpu.sync_copy(z_d, acc_sh.at[pl.ds(base_row, rpt)])
    if cnt_sh is not None:
        pltpu.sync_copy(z_c, cnt_sh.at[pl.ds(base_row, rpt)])
        pltpu.sync_copy(ones_h, ones_v)

    @pl.when(s == 0)
    def _():
        pltpu.sync_copy(z_d.at[pl.ds(0, tail)],
                        acc_sh.at[pl.ds(_NS * rpt, tail)])
        if cnt_sh is not None:
            pltpu.sync_copy(z_c.at[pl.ds(0, tail)],
                            cnt_sh.at[pl.ds(_NS * rpt, tail)])

    plsc.subcore_barrier()

    pltpu.sync_copy(ei2.at[wid], ei_v.at[0])
    pltpu.async_copy(table.at[ei_v.at[0, 0]], rows_v.at[0], sem0)

    @pl.loop(0, iters, step=4)
    def _(i):
        for b in range(4):
            j = i + b
            cid_nxt = wid + (j + 1) * _NW

            @pl.when(jnp.logical_and(j + 1 < iters, cid_nxt < n_chunks))
            def _():
                pltpu.sync_copy(ei2.at[cid_nxt], ei_v.at[(b + 1) % 4])
                pltpu.async_copy(table.at[ei_v.at[(b + 1) % 4, 0]],
                                 rows_v.at[(b + 1) % 2], sems[(b + 1) % 2])

            @pl.when(wid + j * _NW < n_chunks)
            def _():
                pltpu.make_async_copy(table.at[pl.ds(0, _CHUNK)],
                                      rows_v.at[b % 2], sems[b % 2]).wait()
                pltpu.sync_copy(rows_v.at[b % 2], acc_sh.at[ei_v.at[b, 1]],
                                add=True)
                if cnt_sh is not None:
                    pltpu.async_copy(ones_v, cnt_sh.at[ei_v.at[b, 1]],
                                     sem_c, add=True)

    if cnt_sh is not None:
        done = (n_chunks - wid + _NW - 1) // _NW

        @pl.loop(0, done)
        def _(i):
            pltpu.make_async_copy(ones_v, cnt_sh.at[pl.ds(0, _CHUNK)],
                                  sem_c).wait()

    plsc.subcore_barrier()

    pltpu.sync_copy(acc_sh.at[pl.ds(base_row, rpt)],
                    sum_out.at[pl.ds(c * n + base_row, rpt)])
    if cnt_sh is not None:
        pltpu.sync_copy(cnt_sh.at[pl.ds(base_row, rpt)],
                        cnt_out.at[pl.ds(c * n + base_row, rpt)])

    @pl.when(s == 0)
    def _():
        pltpu.sync_copy(acc_sh.at[pl.ds(_NS * rpt, n - _NS * rpt)],
                        sum_out.at[pl.ds(c * n + _NS * rpt, n - _NS * rpt)])
        if cnt_sh is not None:
            pltpu.sync_copy(cnt_sh.at[pl.ds(_NS * rpt, n - _NS * rpt)],
                            cnt_out.at[pl.ds(c * n + _NS * rpt, n - _NS * rpt)])


def _grid(e):
    n_chunks = -(-e // _CHUNK)
    iters = -(-n_chunks // _NW)
    iters = -(-iters // 4) * 4
    return n_chunks, iters


@functools.lru_cache(maxsize=None)
def _make_segsum_count(n, e, d):
    n_chunks, iters = _grid(e)
    mesh = plsc.VectorSubcoreMesh(core_axis_name="c", subcore_axis_name="s")

    @functools.partial(
        pl.kernel,
        out_type=(jax.ShapeDtypeStruct((_NC * n, d), jnp.float32),
                  jax.ShapeDtypeStruct((_NC * n, 16), jnp.float32)),
        mesh=mesh,
        scratch_types=[
            pltpu.VMEM((4, 2, _CHUNK), jnp.int32),
            pltpu.VMEM((2, _CHUNK, d), jnp.float32),
            pltpu.VMEM((_CHUNK, 16), jnp.float32),
            pltpu.VMEM_SHARED((n + _PAD, d), jnp.float32),
            pltpu.VMEM_SHARED((n + _PAD, 16), jnp.float32),
            pltpu.SemaphoreType.DMA,
            pltpu.SemaphoreType.DMA,
            pltpu.SemaphoreType.DMA,
        ],
        compiler_params=pltpu.CompilerParams(use_tc_tiling_on_sc=False),
    )
    def seg(table, ei2, z_d, z_c, ones_h, sum_out, cnt_out,
            ei_v, rows_v, ones_v, acc_sh, cnt_sh, sem0, sem1, sem_c):
        _seg_body(n, d, n_chunks, iters, table, ei2, z_d, sum_out,
                  ei_v, rows_v, acc_sh, sem0, sem1, sem_c=sem_c,
                  z_c=z_c, ones_h=ones_h, cnt_out=cnt_out,
                  ones_v=ones_v, cnt_sh=cnt_sh)

    return seg


@functools.lru_cache(maxsize=None)
def _make_segsum(n, e, d):
    n_chunks, iters = _grid(e)
    mesh = plsc.VectorSubcoreMesh(core_axis_name="c", subcore_axis_name="s")

    @functools.partial(
        pl.kernel,
        out_type=jax.ShapeDtypeStruct((_NC * n, d), jnp.float32),
        mesh=mesh,
        scratch_types=[
            pltpu.VMEM((4, 2, _CHUNK), jnp.int32),
            pltpu.VMEM((2, _CHUNK, d), jnp.float32),
            pltpu.VMEM_SHARED((n + _PAD, d), jnp.float32),
            pltpu.SemaphoreType.DMA,
            pltpu.SemaphoreType.DMA,
        ],
        compiler_params=pltpu.CompilerParams(use_tc_tiling_on_sc=False),
    )
    def seg(table, ei2, z_d, sum_out, ei_v, rows_v, acc_sh, sem0, sem1):
        _seg_body(n, d, n_chunks, iters, table, ei2, z_d, sum_out,
                  ei_v, rows_v, acc_sh, sem0, sem1)

    return seg


def _mm_body(x_ref, wl_ref, wr_ref, p_ref, xr_ref):
    x = x_ref[...]
    p_ref[...] = jnp.dot(x, wl_ref[...], preferred_element_type=jnp.float32)
    xr_ref[...] = jnp.dot(x, wr_ref[...], preferred_element_type=jnp.float32)


def _matmul2(x, wl, wr):
    n = x.shape[0]
    return pl.pallas_call(
        _mm_body,
        out_shape=(jax.ShapeDtypeStruct((n, wl.shape[1]), jnp.float32),
                   jax.ShapeDtypeStruct((n, wr.shape[1]), jnp.float32)),
    )(x, wl, wr)


def _layer1(sums, cnts, xr, b):
    n = xr.shape[0]

    def body(s_ref, c_ref, xr_ref, b_ref, o_ref):
        sarr = s_ref[...]
        carr = c_ref[...]
        sm = sarr[:n] + sarr[n:]
        cnt = carr[:n, 0:1] + carr[n:, 0:1]
        o_ref[...] = jnp.maximum(sm / jnp.maximum(cnt, 1.0) + b_ref[...]
                                 + xr_ref[...], 0.0)

    return pl.pallas_call(
        body,
        out_shape=jax.ShapeDtypeStruct(xr.shape, jnp.float32),
    )(sums, cnts, xr, b)


def _layer2(sums, cnts, h, wl, wr, b):
    n = h.shape[0]

    def body(s_ref, c_ref, h_ref, wl_ref, wr_ref, b_ref, o_ref):
        sarr = s_ref[...]
        carr = c_ref[...]
        sm = sarr[:n] + sarr[n:]
        cnt = carr[:n, 0:1] + carr[n:, 0:1]
        a2 = sm / jnp.maximum(cnt, 1.0)
        o = (jnp.dot(a2, wl_ref[...], preferred_element_type=jnp.float32)
             + jnp.dot(h_ref[...], wr_ref[...],
                       preferred_element_type=jnp.float32)
             + b_ref[...])
        m = jnp.max(o, axis=1, keepdims=True)
        lse = jnp.log(jnp.sum(jnp.exp(o - m), axis=1, keepdims=True)) + m
        o_ref[...] = o - lse

    return pl.pallas_call(
        body,
        out_shape=jax.ShapeDtypeStruct((n, wl.shape[1]), jnp.float32),
    )(sums, cnts, h, wl, wr, b)


def kernel(x, edge_index, W1l, b1l, W1r, W2l, b2l, W2r):
    n, _ = x.shape
    d_hid = W1l.shape[1]
    e = edge_index.shape[1]
    src = edge_index[0]
    dst = edge_index[1]

    e_pad = -(-e // _CHUNK) * _CHUNK
    if e_pad != e:
        src = jnp.concatenate([src, jnp.zeros((e_pad - e,), jnp.int32)])
        dst = jnp.concatenate([dst, jnp.full((e_pad - e,), n, jnp.int32)])
    ei2 = jnp.stack([src.reshape(-1, _CHUNK), dst.reshape(-1, _CHUNK)], axis=1)

    p, xr = _matmul2(x, W1l, W1r)

    rpt = (n // (_NS * 8)) * 8
    z_d = jnp.zeros((rpt, d_hid), jnp.float32)
    z_c = jnp.zeros((rpt, 16), jnp.float32)
    ones_h = jnp.ones((_CHUNK, 16), jnp.float32)

    sums1, cnts = _make_segsum_count(n, e, d_hid)(p, ei2, z_d, z_c, ones_h)
    h = _layer1(sums1, cnts, xr, b1l.reshape(1, -1))
    sums2 = _make_segsum(n, e, d_hid)(h, ei2, z_d)
    return _layer2(sums2, cnts, h, W2l, W2r, b2l.reshape(1, -1))
